# baseline (speedup 1.0000x reference)
import functools

import jax
import jax.numpy as jnp
from jax import lax
from jax.experimental import pallas as pl
from jax.experimental.pallas import tpu as pltpu
from jax.experimental.pallas import tpu_sc as plsc

VOCAB = 256
DIM = 768
GATHER_ROWS = 80
N_HALVES = 2


def _make_kernel(batch: int, seq: int):
  info = plsc.get_sparse_core_info()
  nc, ns = info.num_cores, info.num_subcores
  nw = nc * ns
  per_w = batch // nw
  half = per_w // N_HALVES
  n_pairs = half // 2
  assert batch % (2 * N_HALVES * nw) == 0
  aligned = (seq // 8) * 8

  mesh = plsc.VectorSubcoreMesh(core_axis_name="c", subcore_axis_name="s")

  @functools.partial(
      pl.kernel,
      out_type=jax.ShapeDtypeStruct((batch, seq, DIM), jnp.float32),
      mesh=mesh,
      scratch_types=[
          pltpu.VMEM((half * GATHER_ROWS,), jnp.int32),
          pltpu.VMEM((GATHER_ROWS, DIM), jnp.float32),
          pltpu.VMEM((GATHER_ROWS, DIM), jnp.float32),
          pltpu.SemaphoreType.DMA,
          pltpu.SemaphoreType.DMA,
          pltpu.SemaphoreType.DMA,
          pltpu.SemaphoreType.DMA,
      ],
      compiler_params=pltpu.CompilerParams(use_tc_tiling_on_sc=True),
  )
  def gather_kernel(ids_hbm, table_hbm, out_hbm,
                    idx_blk, rows0, rows1, sem0, sem1, semw0, semw1):
    c = lax.axis_index("c")
    s = lax.axis_index("s")
    wid = s * nc + c
    r0 = wid * per_w

    def glist(j):
      return idx_blk.at[pl.ds(j * GATHER_ROWS, GATHER_ROWS)]

    def panel_copies(rows_v, b, sem):
      cps = [pltpu.make_async_copy(rows_v.at[pl.ds(0, aligned)],
                                   out_hbm.at[b, pl.ds(0, aligned)], sem)]
      for t in range(aligned, seq):
        cps.append(pltpu.make_async_copy(
            rows_v.at[pl.ds(t, 1)], out_hbm.at[b, pl.ds(t, 1)], sem))
      return cps

    def fire_panel(rows_v, b, sem):
      for cp in panel_copies(rows_v, b, sem):
        cp.start()

    def drain_panel(rows_v, b, sem):
      for cp in panel_copies(rows_v, b, sem):
        cp.wait()

    for h in range(N_HALVES):
      b0 = r0 + h * half
      pltpu.sync_copy(ids_hbm.at[pl.ds(b0 * GATHER_ROWS, half * GATHER_ROWS)],
                      idx_blk)
      pltpu.async_copy(table_hbm.at[glist(0)], rows0, sem0)
      pltpu.async_copy(table_hbm.at[glist(1)], rows1, sem1)

      def body(i, carry):
        j = 2 * i
        b = b0 + j
        pltpu.make_async_copy(table_hbm.at[glist(j)], rows0, sem0).wait()
        fire_panel(rows0, b, semw0)
        pltpu.make_async_copy(table_hbm.at[glist(j + 1)], rows1, sem1).wait()
        fire_panel(rows1, b + 1, semw1)

        @pl.when(i < n_pairs - 1)
        def _():
          drain_panel(rows0, b, semw0)
          pltpu.async_copy(table_hbm.at[glist(j + 2)], rows0, sem0)
          drain_panel(rows1, b + 1, semw1)
          pltpu.async_copy(table_hbm.at[glist(j + 3)], rows1, sem1)

        @pl.when(i == n_pairs - 1)
        def _():
          drain_panel(rows0, b, semw0)
          drain_panel(rows1, b + 1, semw1)

        return carry

      lax.fori_loop(0, n_pairs, body, 0)

  return gather_kernel


def kernel(token_ids, table):
  b, t = token_ids.shape
  ids_pad = jnp.pad(token_ids.astype(jnp.int32),
                    ((0, 0), (0, GATHER_ROWS - t)))
  flat = ids_pad.reshape(b * GATHER_ROWS)
  return _make_kernel(b, t)(flat, table)


# speedup vs baseline: 1.2733x; 1.2733x over previous
import functools

import jax
import jax.numpy as jnp
from jax import lax
from jax.experimental import pallas as pl
from jax.experimental.pallas import tpu as pltpu
from jax.experimental.pallas import tpu_sc as plsc

VOCAB = 256
DIM = 768
CHUNK = 64


def _make_kernel(num_tokens: int):
  info = plsc.get_sparse_core_info()
  nc, ns = info.num_cores, info.num_subcores
  nw = nc * ns
  assert num_tokens % (nw * 2 * CHUNK) == 0
  per_w = num_tokens // nw
  n_pairs = per_w // (2 * CHUNK)

  mesh = plsc.VectorSubcoreMesh(core_axis_name="c", subcore_axis_name="s")

  @functools.partial(
      pl.kernel,
      out_type=jax.ShapeDtypeStruct((num_tokens, DIM), jnp.float32),
      mesh=mesh,
      scratch_types=[
          pltpu.VMEM((per_w,), jnp.int32),
          pltpu.VMEM((CHUNK, DIM), jnp.float32),
          pltpu.VMEM((CHUNK, DIM), jnp.float32),
          pltpu.SemaphoreType.DMA,
          pltpu.SemaphoreType.DMA,
      ],
      compiler_params=pltpu.CompilerParams(use_tc_tiling_on_sc=True),
  )
  def gather_kernel(ids_hbm, table_hbm, out_hbm,
                    idx_blk, rows0, rows1, sem0, sem1):
    c = lax.axis_index("c")
    s = lax.axis_index("s")
    wid = s * nc + c
    base_w = wid * per_w

    pltpu.sync_copy(ids_hbm.at[pl.ds(base_w, per_w)], idx_blk)

    pltpu.async_copy(table_hbm.at[idx_blk.at[pl.ds(0, CHUNK)]], rows0, sem0)

    def body(i, carry):
      o = i * 2 * CHUNK
      pltpu.async_copy(
          table_hbm.at[idx_blk.at[pl.ds(o + CHUNK, CHUNK)]], rows1, sem1)
      pltpu.make_async_copy(
          table_hbm.at[idx_blk.at[pl.ds(o, CHUNK)]], rows0, sem0).wait()
      pltpu.sync_copy(rows0, out_hbm.at[pl.ds(base_w + o, CHUNK)])

      @pl.when(i < n_pairs - 1)
      def _():
        pltpu.async_copy(
            table_hbm.at[idx_blk.at[pl.ds(o + 2 * CHUNK, CHUNK)]], rows0, sem0)

      pltpu.make_async_copy(
          table_hbm.at[idx_blk.at[pl.ds(o + CHUNK, CHUNK)]], rows1, sem1).wait()
      pltpu.sync_copy(rows1, out_hbm.at[pl.ds(base_w + o + CHUNK, CHUNK)])
      return carry

    lax.fori_loop(0, n_pairs, body, 0)

  return gather_kernel


GRP = 16


def _relayout_body(x_ref, y_ref):
  seq = y_ref.shape[1]
  for j in range(GRP):
    y_ref[j] = x_ref[pl.ds(j * seq, seq), :]


def _relayout(x, batch: int, seq: int):
  return pl.pallas_call(
      _relayout_body,
      out_shape=jax.ShapeDtypeStruct((batch, seq, DIM), jnp.float32),
      in_specs=[pl.BlockSpec((GRP * seq, DIM), lambda g: (g, 0))],
      out_specs=pl.BlockSpec((GRP, seq, DIM), lambda g: (g, 0, 0)),
      grid=(batch // GRP,),
  )(x)


def kernel(token_ids, table):
  b, t = token_ids.shape
  flat = token_ids.reshape(b * t).astype(jnp.int32)
  out = _make_kernel(b * t)(flat, table)
  return _relayout(out, b, t)


# speedup vs baseline: 1.2782x; 1.0038x over previous
import functools

import jax
import jax.numpy as jnp
from jax import lax
from jax.experimental import pallas as pl
from jax.experimental.pallas import tpu as pltpu
from jax.experimental.pallas import tpu_sc as plsc

VOCAB = 256
DIM = 768
CHUNK = 64


def _make_kernel(num_tokens: int):
  info = plsc.get_sparse_core_info()
  nc, ns = info.num_cores, info.num_subcores
  nw = nc * ns
  assert num_tokens % (nw * 2 * CHUNK) == 0
  per_w = num_tokens // nw
  n_pairs = per_w // (2 * CHUNK)

  mesh = plsc.VectorSubcoreMesh(core_axis_name="c", subcore_axis_name="s")

  @functools.partial(
      pl.kernel,
      out_type=jax.ShapeDtypeStruct((num_tokens, DIM), jnp.float32),
      mesh=mesh,
      scratch_types=[
          pltpu.VMEM((per_w,), jnp.int32),
          pltpu.VMEM((CHUNK, DIM), jnp.float32),
          pltpu.VMEM((CHUNK, DIM), jnp.float32),
          pltpu.SemaphoreType.DMA,
          pltpu.SemaphoreType.DMA,
      ],
      compiler_params=pltpu.CompilerParams(use_tc_tiling_on_sc=True),
  )
  def gather_kernel(ids_hbm, table_hbm, out_hbm,
                    idx_blk, rows0, rows1, sem0, sem1):
    c = lax.axis_index("c")
    s = lax.axis_index("s")
    wid = s * nc + c
    base_w = wid * per_w

    pltpu.sync_copy(ids_hbm.at[pl.ds(base_w, per_w)], idx_blk)

    pltpu.async_copy(table_hbm.at[idx_blk.at[pl.ds(0, CHUNK)]], rows0, sem0)

    def body(i, carry):
      o = i * 2 * CHUNK
      pltpu.async_copy(
          table_hbm.at[idx_blk.at[pl.ds(o + CHUNK, CHUNK)]], rows1, sem1)
      pltpu.make_async_copy(
          table_hbm.at[idx_blk.at[pl.ds(o, CHUNK)]], rows0, sem0).wait()
      pltpu.sync_copy(rows0, out_hbm.at[pl.ds(base_w + o, CHUNK)])

      @pl.when(i < n_pairs - 1)
      def _():
        pltpu.async_copy(
            table_hbm.at[idx_blk.at[pl.ds(o + 2 * CHUNK, CHUNK)]], rows0, sem0)

      pltpu.make_async_copy(
          table_hbm.at[idx_blk.at[pl.ds(o + CHUNK, CHUNK)]], rows1, sem1).wait()
      pltpu.sync_copy(rows1, out_hbm.at[pl.ds(base_w + o + CHUNK, CHUNK)])
      return carry

    lax.fori_loop(0, n_pairs, body, 0)

  return gather_kernel


GRP = 32


def _relayout_body(x_ref, y_ref):
  seq = y_ref.shape[1]
  for j in range(GRP):
    y_ref[j] = x_ref[pl.ds(j * seq, seq), :]


def _relayout(x, batch: int, seq: int):
  return pl.pallas_call(
      _relayout_body,
      out_shape=jax.ShapeDtypeStruct((batch, seq, DIM), jnp.float32),
      in_specs=[pl.BlockSpec((GRP * seq, DIM), lambda g: (g, 0))],
      out_specs=pl.BlockSpec((GRP, seq, DIM), lambda g: (g, 0, 0)),
      grid=(batch // GRP,),
  )(x)


def kernel(token_ids, table):
  b, t = token_ids.shape
  flat = token_ids.reshape(b * t).astype(jnp.int32)
  out = _make_kernel(b * t)(flat, table)
  return _relayout(out, b, t)


# speedup vs baseline: 1.2857x; 1.0058x over previous
import functools

import jax
import jax.numpy as jnp
from jax import lax
from jax.experimental import pallas as pl
from jax.experimental.pallas import tpu as pltpu
from jax.experimental.pallas import tpu_sc as plsc

VOCAB = 256
DIM = 768
CHUNK = 56
GRP = 16
K_CHUNKS = 2


def _make_kernel(num_tokens: int):
  info = plsc.get_sparse_core_info()
  nc, ns = info.num_cores, info.num_subcores
  nw = nc * ns
  assert num_tokens % (nw * 2 * CHUNK) == 0
  per_w = num_tokens // nw
  n_pairs = per_w // (2 * CHUNK)

  mesh = plsc.VectorSubcoreMesh(core_axis_name="c", subcore_axis_name="s")

  @functools.partial(
      pl.kernel,
      out_type=jax.ShapeDtypeStruct((num_tokens, DIM), jnp.float32),
      mesh=mesh,
      scratch_types=[
          pltpu.VMEM((per_w,), jnp.int32),
          pltpu.VMEM((CHUNK, DIM), jnp.float32),
          pltpu.VMEM((CHUNK, DIM), jnp.float32),
          pltpu.SemaphoreType.DMA,
          pltpu.SemaphoreType.DMA,
      ],
      compiler_params=pltpu.CompilerParams(use_tc_tiling_on_sc=True),
  )
  def gather_kernel(ids_hbm, table_hbm, out_hbm,
                    idx_blk, rows0, rows1, sem0, sem1):
    c = lax.axis_index("c")
    s = lax.axis_index("s")
    wid = s * nc + c
    base_w = wid * per_w

    pltpu.sync_copy(ids_hbm.at[pl.ds(base_w, per_w)], idx_blk)

    pltpu.async_copy(table_hbm.at[idx_blk.at[pl.ds(0, CHUNK)]], rows0, sem0)

    def body(i, carry):
      o = i * 2 * CHUNK
      pltpu.async_copy(
          table_hbm.at[idx_blk.at[pl.ds(o + CHUNK, CHUNK)]], rows1, sem1)
      pltpu.make_async_copy(
          table_hbm.at[idx_blk.at[pl.ds(o, CHUNK)]], rows0, sem0).wait()
      pltpu.sync_copy(rows0, out_hbm.at[pl.ds(base_w + o, CHUNK)])

      @pl.when(i < n_pairs - 1)
      def _():
        pltpu.async_copy(
            table_hbm.at[idx_blk.at[pl.ds(o + 2 * CHUNK, CHUNK)]], rows0, sem0)

      pltpu.make_async_copy(
          table_hbm.at[idx_blk.at[pl.ds(o + CHUNK, CHUNK)]], rows1, sem1).wait()
      pltpu.sync_copy(rows1, out_hbm.at[pl.ds(base_w + o + CHUNK, CHUNK)])
      return carry

    lax.fori_loop(0, n_pairs, body, 0)

  return gather_kernel


def _relayout_chunk(x, y_prev, batch: int, seq: int, blk_base: int,
                    n_blocks: int):
  def body(*refs):
    x_ref, y_ref = refs[-2], refs[-1]
    for j in range(GRP):
      y_ref[j] = x_ref[pl.ds(j * seq, seq), :]

  x_spec = pl.BlockSpec((GRP * seq, DIM), lambda g: (g, 0))
  y_spec = pl.BlockSpec((GRP, seq, DIM), lambda g: (blk_base + g, 0, 0))
  out_shape = jax.ShapeDtypeStruct((batch, seq, DIM), jnp.float32)
  if y_prev is None:
    return pl.pallas_call(
        body, out_shape=out_shape, grid=(n_blocks,),
        in_specs=[x_spec], out_specs=y_spec,
    )(x)
  return pl.pallas_call(
      body, out_shape=out_shape, grid=(n_blocks,),
      in_specs=[pl.BlockSpec(memory_space=pl.ANY), x_spec],
      out_specs=y_spec,
      input_output_aliases={0: 0},
  )(y_prev, x)


def kernel(token_ids, table):
  b, t = token_ids.shape
  flat = token_ids.reshape(b * t).astype(jnp.int32)
  bk = b // K_CHUNKS
  nk = bk * t
  sc_gather = _make_kernel(nk)
  y = None
  for k in range(K_CHUNKS):
    xk = sc_gather(lax.slice(flat, (k * nk,), ((k + 1) * nk,)), table)
    y = _relayout_chunk(xk, y, b, t, k * (bk // GRP), bk // GRP)
  return y


# speedup vs baseline: 1.2930x; 1.0057x over previous
import functools

import jax
import jax.numpy as jnp
from jax import lax
from jax.experimental import pallas as pl
from jax.experimental.pallas import tpu as pltpu
from jax.experimental.pallas import tpu_sc as plsc

VOCAB = 256
DIM = 768
CHUNK = 56
GRP = 16
K_CHUNKS = 4


def _make_kernel(num_tokens: int):
  info = plsc.get_sparse_core_info()
  nc, ns = info.num_cores, info.num_subcores
  nw = nc * ns
  assert num_tokens % (nw * 2 * CHUNK) == 0
  per_w = num_tokens // nw
  n_pairs = per_w // (2 * CHUNK)

  mesh = plsc.VectorSubcoreMesh(core_axis_name="c", subcore_axis_name="s")

  @functools.partial(
      pl.kernel,
      out_type=jax.ShapeDtypeStruct((num_tokens, DIM), jnp.float32),
      mesh=mesh,
      scratch_types=[
          pltpu.VMEM((per_w,), jnp.int32),
          pltpu.VMEM((CHUNK, DIM), jnp.float32),
          pltpu.VMEM((CHUNK, DIM), jnp.float32),
          pltpu.SemaphoreType.DMA,
          pltpu.SemaphoreType.DMA,
      ],
      compiler_params=pltpu.CompilerParams(use_tc_tiling_on_sc=True),
  )
  def gather_kernel(ids_hbm, table_hbm, out_hbm,
                    idx_blk, rows0, rows1, sem0, sem1):
    c = lax.axis_index("c")
    s = lax.axis_index("s")
    wid = s * nc + c
    base_w = wid * per_w

    pltpu.sync_copy(ids_hbm.at[pl.ds(base_w, per_w)], idx_blk)

    pltpu.async_copy(table_hbm.at[idx_blk.at[pl.ds(0, CHUNK)]], rows0, sem0)

    def body(i, carry):
      o = i * 2 * CHUNK
      pltpu.async_copy(
          table_hbm.at[idx_blk.at[pl.ds(o + CHUNK, CHUNK)]], rows1, sem1)
      pltpu.make_async_copy(
          table_hbm.at[idx_blk.at[pl.ds(o, CHUNK)]], rows0, sem0).wait()
      pltpu.sync_copy(rows0, out_hbm.at[pl.ds(base_w + o, CHUNK)])

      @pl.when(i < n_pairs - 1)
      def _():
        pltpu.async_copy(
            table_hbm.at[idx_blk.at[pl.ds(o + 2 * CHUNK, CHUNK)]], rows0, sem0)

      pltpu.make_async_copy(
          table_hbm.at[idx_blk.at[pl.ds(o + CHUNK, CHUNK)]], rows1, sem1).wait()
      pltpu.sync_copy(rows1, out_hbm.at[pl.ds(base_w + o + CHUNK, CHUNK)])
      return carry

    lax.fori_loop(0, n_pairs, body, 0)

  return gather_kernel


def _relayout_chunk(x, y_prev, batch: int, seq: int, blk_base: int,
                    n_blocks: int):
  def body(*refs):
    x_ref, y_ref = refs[-2], refs[-1]
    for j in range(GRP):
      y_ref[j] = x_ref[pl.ds(j * seq, seq), :]

  x_spec = pl.BlockSpec((GRP * seq, DIM), lambda g: (g, 0))
  y_spec = pl.BlockSpec((GRP, seq, DIM), lambda g: (blk_base + g, 0, 0))
  out_shape = jax.ShapeDtypeStruct((batch, seq, DIM), jnp.float32)
  if y_prev is None:
    return pl.pallas_call(
        body, out_shape=out_shape, grid=(n_blocks,),
        in_specs=[x_spec], out_specs=y_spec,
    )(x)
  return pl.pallas_call(
      body, out_shape=out_shape, grid=(n_blocks,),
      in_specs=[pl.BlockSpec(memory_space=pl.ANY), x_spec],
      out_specs=y_spec,
      input_output_aliases={0: 0},
  )(y_prev, x)


def kernel(token_ids, table):
  b, t = token_ids.shape
  flat = token_ids.reshape(b * t).astype(jnp.int32)
  bk = b // K_CHUNKS
  nk = bk * t
  sc_gather = _make_kernel(nk)
  y = None
  for k in range(K_CHUNKS):
    xk = sc_gather(lax.slice(flat, (k * nk,), ((k + 1) * nk,)), table)
    y = _relayout_chunk(xk, y, b, t, k * (bk // GRP), bk // GRP)
  return y
